# Initial kernel scaffold; baseline (speedup 1.0000x reference)
import jax
import jax.numpy as jnp
from jax.experimental import pallas as pl


def kernel(x, x1):
    raise NotImplementedError("write your pallas kernel here")


# speedup vs baseline: 3.8731x; 3.8731x over previous
import jax
import jax.numpy as jnp
from jax import lax
from jax.experimental import pallas as pl
from jax.experimental.pallas import tpu as pltpu
from jax.experimental.pallas import tpu_sc as plsc

_B, _C, _H, _W = 8, 96, 112, 112
_STRIDE, _KS = 2, 2
_HOUT = (_H - 1) * _STRIDE + _KS
_WOUT = (_W - 1) * _STRIDE + _KS
_NPLANES = _B * _C
_IN_PLANE = _H * _W
_OUT_PLANE = _HOUT * _WOUT
_NWORKERS = 32
_PPW = _NPLANES // _NWORKERS
_LANES = 16


def _unpool_body(ks_hbm, vs_hbm, out_hbm, key_v, val_v, plane_v, in_sem):
    cid = lax.axis_index("c")
    sid = lax.axis_index("s")
    wid = sid * 2 + cid

    zero = jnp.zeros((_LANES,), jnp.float32)
    sentinel = jnp.full((_LANES,), -1, jnp.int32)

    def plane_step(j, carry):
        p = wid * _PPW + j
        kcopy = pltpu.async_copy(ks_hbm.at[p], key_v.at[pl.ds(0, _IN_PLANE)],
                                 in_sem)
        vcopy = pltpu.async_copy(vs_hbm.at[p], val_v, in_sem)

        def zero_step(i, c):
            plane_v[pl.ds(pl.multiple_of(i * _LANES, _LANES), _LANES)] = zero
            return c

        lax.fori_loop(0, _OUT_PLANE // _LANES, zero_step, 0, unroll=8)

        kcopy.wait()
        vcopy.wait()
        key_v[pl.ds(_IN_PLANE, _LANES)] = sentinel

        base = p * _OUT_PLANE

        def scatter_step(i, c):
            off = pl.ds(pl.multiple_of(i * _LANES, _LANES), _LANES)
            kv = key_v[off]
            nxt = key_v[pl.ds(i * _LANES + 1, _LANES)]
            vv = val_v[off]
            keep = kv != nxt
            liv = kv - base
            plsc.store_scatter(plane_v, [liv], vv, mask=keep)
            return c

        lax.fori_loop(0, _IN_PLANE // _LANES, scatter_step, 0, unroll=4)

        pltpu.sync_copy(plane_v, out_hbm.at[p])
        return carry

    lax.fori_loop(0, _PPW, plane_step, 0)


def kernel(x, x1):
    rows = jnp.arange(_NPLANES, dtype=jnp.int32) * _OUT_PLANE
    keys = (x1.reshape(_NPLANES, _IN_PLANE) + rows[:, None]).reshape(-1)
    vals = x.reshape(-1)
    ks, vs = lax.sort((keys, vals), dimension=0, is_stable=False, num_keys=1)
    ksr = ks.reshape(_NPLANES, _IN_PLANE)
    vsr = vs.reshape(_NPLANES, _IN_PLANE)

    mesh = plsc.VectorSubcoreMesh(core_axis_name="c", subcore_axis_name="s")
    f = pl.kernel(
        _unpool_body,
        mesh=mesh,
        out_type=jax.ShapeDtypeStruct((_NPLANES, _OUT_PLANE), jnp.float32),
        compiler_params=pltpu.CompilerParams(needs_layout_passes=False),
        scratch_types=[
            pltpu.VMEM((_IN_PLANE + _LANES,), jnp.int32),
            pltpu.VMEM((_IN_PLANE,), jnp.float32),
            pltpu.VMEM((_OUT_PLANE,), jnp.float32),
            pltpu.SemaphoreType.DMA,
        ],
    )
    out = f(ksr, vsr)
    return out.reshape(_B, _C, _HOUT, _WOUT)


# speedup vs baseline: 3.9049x; 1.0082x over previous
import jax
import jax.numpy as jnp
from jax import lax
from jax.experimental import pallas as pl
from jax.experimental.pallas import tpu as pltpu
from jax.experimental.pallas import tpu_sc as plsc

_B, _C, _H, _W = 8, 96, 112, 112
_STRIDE, _KS = 2, 2
_HOUT = (_H - 1) * _STRIDE + _KS
_WOUT = (_W - 1) * _STRIDE + _KS
_NPLANES = _B * _C
_IN_PLANE = _H * _W
_OUT_PLANE = _HOUT * _WOUT
_NWORKERS = 32
_PPW = _NPLANES // _NWORKERS
_LANES = 16


def _unpool_body(ks_hbm, vs_hbm, out_hbm, key_v, val_v, plane0_v, plane1_v,
                 in_sem, out_sem0, out_sem1):
    cid = lax.axis_index("c")
    sid = lax.axis_index("s")
    wid = sid * 2 + cid

    zero = jnp.zeros((_LANES,), jnp.float32)
    sentinel = jnp.full((_LANES,), -1, jnp.int32)
    planes = (plane0_v, plane1_v)
    out_sems = (out_sem0, out_sem1)

    def do_plane(p, plane_v, out_sem, first, last):
        kcopy = pltpu.async_copy(
            ks_hbm.at[pl.ds(p * _IN_PLANE, _IN_PLANE)],
            key_v.at[pl.ds(0, _IN_PLANE)], in_sem)
        vcopy = pltpu.async_copy(
            vs_hbm.at[pl.ds(p * _IN_PLANE, _IN_PLANE)], val_v, in_sem)

        @pl.when(jnp.logical_not(first))
        def _():
            prev_base = (p - 2) * _OUT_PLANE
            pltpu.make_async_copy(
                plane_v, out_hbm.at[pl.ds(prev_base, _OUT_PLANE)],
                out_sem).wait()

        def zero_step(i, c):
            plane_v[pl.ds(pl.multiple_of(i * _LANES, _LANES), _LANES)] = zero
            return c

        lax.fori_loop(0, _OUT_PLANE // _LANES, zero_step, 0, unroll=8)

        kcopy.wait()
        vcopy.wait()
        key_v[pl.ds(_IN_PLANE, _LANES)] = sentinel

        base = p * _OUT_PLANE

        def scatter_step(i, c):
            off = pl.ds(pl.multiple_of(i * _LANES, _LANES), _LANES)
            kv = key_v[off]
            nxt = key_v[pl.ds(i * _LANES + 1, _LANES)]
            vv = val_v[off]
            keep = kv != nxt
            liv = kv - base
            plsc.store_scatter(plane_v, [liv], vv, mask=keep)
            return c

        lax.fori_loop(0, _IN_PLANE // _LANES, scatter_step, 0, unroll=4)

        ocopy = pltpu.async_copy(
            plane_v, out_hbm.at[pl.ds(base, _OUT_PLANE)], out_sem)

        @pl.when(last)
        def _():
            ocopy.wait()

    def pair_step(jj, carry):
        for b in range(2):
            j = jj * 2 + b
            p = wid * _PPW + j
            do_plane(p, planes[b], out_sems[b],
                     first=(j <= 1), last=(j >= _PPW - 2))
        return carry

    lax.fori_loop(0, _PPW // 2, pair_step, 0)


def kernel(x, x1):
    rows = jnp.arange(_NPLANES, dtype=jnp.int32) * _OUT_PLANE
    keys = (x1.reshape(_NPLANES, _IN_PLANE) + rows[:, None]).reshape(-1)
    vals = x.reshape(-1)
    ks, vs = lax.sort((keys, vals), dimension=0, is_stable=False, num_keys=1)

    mesh = plsc.VectorSubcoreMesh(core_axis_name="c", subcore_axis_name="s")
    f = pl.kernel(
        _unpool_body,
        mesh=mesh,
        out_type=jax.ShapeDtypeStruct((_NPLANES * _OUT_PLANE,), jnp.float32),
        compiler_params=pltpu.CompilerParams(needs_layout_passes=False),
        scratch_types=[
            pltpu.VMEM((_IN_PLANE + _LANES,), jnp.int32),
            pltpu.VMEM((_IN_PLANE,), jnp.float32),
            pltpu.VMEM((_OUT_PLANE,), jnp.float32),
            pltpu.VMEM((_OUT_PLANE,), jnp.float32),
            pltpu.SemaphoreType.DMA,
            pltpu.SemaphoreType.DMA,
            pltpu.SemaphoreType.DMA,
        ],
    )
    out = f(ks, vs)
    return out.reshape(_B, _C, _HOUT, _WOUT)


# speedup vs baseline: 3.9068x; 1.0005x over previous
import jax
import jax.numpy as jnp
from jax import lax
from jax.experimental import pallas as pl
from jax.experimental.pallas import tpu as pltpu
from jax.experimental.pallas import tpu_sc as plsc

_B, _C, _H, _W = 8, 96, 112, 112
_STRIDE, _KS = 2, 2
_HOUT = (_H - 1) * _STRIDE + _KS
_WOUT = (_W - 1) * _STRIDE + _KS
_NPLANES = _B * _C
_IN_PLANE = _H * _W
_OUT_PLANE = _HOUT * _WOUT
_NWORKERS = 32
_PPW = _NPLANES // _NWORKERS
_LANES = 16


def _unpool_body(ks_hbm, vs_hbm, out_hbm, key_v, val_v, plane0_v, plane1_v,
                 in_sem, out_sem0, out_sem1):
    cid = lax.axis_index("c")
    sid = lax.axis_index("s")
    wid = sid * 2 + cid

    zero = jnp.zeros((_LANES,), jnp.float32)
    sentinel = jnp.full((_LANES,), -1, jnp.int32)
    planes = (plane0_v, plane1_v)
    out_sems = (out_sem0, out_sem1)

    def do_plane(p, plane_v, out_sem, first, last):
        kcopy = pltpu.async_copy(
            ks_hbm.at[pl.ds(p * _IN_PLANE, _IN_PLANE)],
            key_v.at[pl.ds(0, _IN_PLANE)], in_sem)
        vcopy = pltpu.async_copy(
            vs_hbm.at[pl.ds(p * _IN_PLANE, _IN_PLANE)], val_v, in_sem)

        @pl.when(jnp.logical_not(first))
        def _():
            prev_base = (p - 2) * _OUT_PLANE
            pltpu.make_async_copy(
                plane_v, out_hbm.at[pl.ds(prev_base, _OUT_PLANE)],
                out_sem).wait()

        def zero_step(i, c):
            plane_v[pl.ds(pl.multiple_of(i * _LANES, _LANES), _LANES)] = zero
            return c

        lax.fori_loop(0, _OUT_PLANE // _LANES, zero_step, 0, unroll=16)

        kcopy.wait()
        vcopy.wait()
        key_v[pl.ds(_IN_PLANE, _LANES)] = sentinel

        base = p * _OUT_PLANE

        def scatter_step(i, c):
            off = pl.ds(pl.multiple_of(i * _LANES, _LANES), _LANES)
            kv = key_v[off]
            nxt = key_v[pl.ds(i * _LANES + 1, _LANES)]
            vv = val_v[off]
            keep = kv != nxt
            liv = kv - base
            plsc.store_scatter(plane_v, [liv], vv, mask=keep)
            return c

        lax.fori_loop(0, _IN_PLANE // _LANES, scatter_step, 0, unroll=8)

        ocopy = pltpu.async_copy(
            plane_v, out_hbm.at[pl.ds(base, _OUT_PLANE)], out_sem)

        @pl.when(last)
        def _():
            ocopy.wait()

    def pair_step(jj, carry):
        for b in range(2):
            j = jj * 2 + b
            p = wid * _PPW + j
            do_plane(p, planes[b], out_sems[b],
                     first=(j <= 1), last=(j >= _PPW - 2))
        return carry

    lax.fori_loop(0, _PPW // 2, pair_step, 0)


def kernel(x, x1):
    rows = jnp.arange(_NPLANES, dtype=jnp.int32) * _OUT_PLANE
    keys = (x1.reshape(_NPLANES, _IN_PLANE) + rows[:, None]).reshape(-1)
    vals = x.reshape(-1)
    ks, vs = lax.sort((keys, vals), dimension=0, is_stable=False, num_keys=1)

    mesh = plsc.VectorSubcoreMesh(core_axis_name="c", subcore_axis_name="s")
    f = pl.kernel(
        _unpool_body,
        mesh=mesh,
        out_type=jax.ShapeDtypeStruct((_NPLANES * _OUT_PLANE,), jnp.float32),
        compiler_params=pltpu.CompilerParams(needs_layout_passes=False),
        scratch_types=[
            pltpu.VMEM((_IN_PLANE + _LANES,), jnp.int32),
            pltpu.VMEM((_IN_PLANE,), jnp.float32),
            pltpu.VMEM((_OUT_PLANE,), jnp.float32),
            pltpu.VMEM((_OUT_PLANE,), jnp.float32),
            pltpu.SemaphoreType.DMA,
            pltpu.SemaphoreType.DMA,
            pltpu.SemaphoreType.DMA,
        ],
    )
    out = f(ks, vs)
    return out.reshape(_B, _C, _HOUT, _WOUT)
